# baseline (speedup 1.0000x reference)
import functools

import jax
import jax.numpy as jnp
from jax import lax
from jax.experimental import pallas as pl
from jax.experimental.pallas import tpu as pltpu
from jax.experimental.pallas import tpu_sc as plsc

HH = 512
RECENT = 512
CACHE = HH + RECENT
T = 4096
SEL = T - RECENT
D = 128
NPAIR = 256
NC, NS = 2, 16
NW = NC * NS
PPW = NPAIR // NW


def _sum_body(attn_ref, hh_ref):
    x = attn_ref[...]
    hh = jnp.sum(x, axis=1)
    hh_ref[...] = lax.bitcast_convert_type(hh, jnp.int32)


def _tc_sum(attn):
    return pl.pallas_call(
        _sum_body,
        grid=(NPAIR // 8,),
        in_specs=[pl.BlockSpec((8, 8, T), lambda i: (i, 0, 0))],
        out_specs=pl.BlockSpec((8, T), lambda i: (i, 0)),
        out_shape=jax.ShapeDtypeStruct((NPAIR, T), jnp.int32),
    )(attn)


def _sc_body(hh_hbm, k_hbm, v_hbm, ko_hbm, vo_hbm, hn_hbm,
             row_v, ca_v, cb_v, idx_v, hn_v, buf_v, gsem, osem):
    wid = lax.axis_index("s") * NC + lax.axis_index("c")
    lane = lax.iota(jnp.int32, 16)

    def pair_body(j, _):
        jc = wid * PPW + j
        do_t = j >= 1
        jcc = jnp.where(j < PPW, jc, jc - 1)
        cpar = j & 1
        tpar = 1 - cpar
        coff = cpar * CACHE
        toff = tpar * CACHE
        base = jcc * T
        base_t = (jc - 1) * T
        orow_t = (jc - 1) * CACHE

        NB = 3
        NT = 16
        gd = [pltpu.make_async_copy(
            (k_hbm if t % 2 == 0 else v_hbm).at[
                pl.ds(base_t + SEL + ((t % 8) // 2) * 128, 128)]
            if t < 8 else
            (k_hbm if t % 2 == 0 else v_hbm).at[
                idx_v.at[pl.ds(toff + ((t % 8) // 2) * 128, 128)]],
            buf_v.at[t % NB], gsem) for t in range(NT)]
        od = [pltpu.make_async_copy(
            buf_v.at[t % NB],
            (ko_hbm if t % 2 == 0 else vo_hbm).at[
                pl.ds(orow_t + ((4 if t < 8 else 0) + (t % 8) // 2) * 128,
                      128)],
            osem) for t in range(NT)]

        def step(t):
            @pl.when(do_t)
            def _():
                if t >= NB:
                    od[t - NB].wait()
                gd[t].start()
                if t >= 1:
                    gd[t - 1].wait()
                    od[t - 1].start()

        step(0)
        step(1)
        pltpu.sync_copy(hh_hbm.at[jcc], row_v)
        step(2)

        def mmx(i, c):
            mn, mx = c
            v = row_v[pl.ds(i * 16, 16)]
            return jnp.minimum(mn, jnp.min(v)), jnp.maximum(mx, jnp.max(v))

        mn, mx = lax.fori_loop(
            0, SEL // 16, mmx,
            (jnp.int32(0x7FFFFFFF), jnp.int32(-0x80000000)))
        rng = mx - mn

        def hib(b, h):
            return jnp.where((rng >> b) > 0, b, h)

        h = lax.fori_loop(0, 31, hib, jnp.int32(0))
        s = 30 - h
        step(3)

        def icp(i, _):
            ca_v[pl.ds(i * 16, 16)] = (row_v[pl.ds(i * 16, 16)] - mn) << s
            return 0

        lax.fori_loop(0, SEL // 16, icp, 0)
        step(4)

        def rnd(t, st):
            ncand, need, prefix = st
            bit = 30 - t
            nv = (ncand + 15) >> 4

            def c1b(i, c):
                v = ca_v[pl.ds(i * 16, 16)]
                valid = (i * 16 + lane) < ncand
                one = jnp.logical_and(((v >> bit) & 1) == 1, valid)
                return c + jnp.sum(one.astype(jnp.int32))

            cnt1 = lax.fori_loop(0, nv, c1b, jnp.int32(0))
            pick1 = cnt1 >= need
            nnext = jnp.where(pick1, cnt1, ncand - cnt1)
            need2 = jnp.where(pick1, need, need - cnt1)
            prefix2 = jnp.where(pick1, prefix | (1 << bit), prefix)

            def sb(i, c):
                v = ca_v[pl.ds(i * 16, 16)]
                valid = (i * 16 + lane) < ncand
                one = ((v >> bit) & 1) == 1
                keepl = jnp.logical_and(valid, one == pick1)
                ki = keepl.astype(jnp.int32)
                pos = c + plsc.cumsum(ki) - 1
                plsc.store_scatter(cb_v, [pos], v, mask=keepl)
                return c + jnp.sum(ki)

            lax.fori_loop(0, nv, sb, jnp.int32(0))

            def cpb(i, _):
                ca_v[pl.ds(i * 16, 16)] = cb_v[pl.ds(i * 16, 16)]
                return 0

            lax.fori_loop(0, (nnext + 15) >> 4, cpb, 0)
            return (nnext, need2, prefix2)

        st = (jnp.int32(SEL), jnp.int32(HH), jnp.int32(0))
        for ri in range(7):
            st = lax.fori_loop(2 * ri, 2 * ri + 2, rnd, st)
            step(5 + ri)
        st = lax.fori_loop(14, 31, rnd, st)
        step(12)
        _, rq, prefix = st
        vstar = (prefix >> s) + mn

        def cb(i, carry):
            ck, ct = carry
            v = row_v[pl.ds(i * 16, 16)]
            gt = v > vstar
            eq = v == vstar
            eqi = eq.astype(jnp.int32)
            tie_rank = ct + plsc.cumsum(eqi)
            keep = jnp.logical_or(gt, jnp.logical_and(eq, tie_rank <= rq))
            kint = keep.astype(jnp.int32)
            pos = coff + ck + plsc.cumsum(kint) - 1
            gidx = base + i * 16 + lane
            plsc.store_scatter(idx_v, [pos], gidx, mask=keep)
            plsc.store_scatter(hn_v, [pos], v, mask=keep)
            return ck + jnp.sum(kint), ct + jnp.sum(eqi)

        lax.fori_loop(0, SEL // 16, cb, (jnp.int32(0), jnp.int32(0)))
        step(13)

        def rb(i, _):
            off = i * 16
            hn_v[pl.ds(coff + HH + off, 16)] = row_v[pl.ds(SEL + off, 16)]
            return 0

        lax.fori_loop(0, RECENT // 16, rb, 0)
        step(14)

        @pl.when(do_t)
        def _():
            pltpu.sync_copy(hn_v.at[pl.ds(toff, CACHE)], hn_hbm.at[jc - 1])

        step(15)

        @pl.when(do_t)
        def _():
            gd[NT - 1].wait()
            od[NT - 1].start()
            for t in range(NT - NB, NT):
                od[t].wait()

        return 0

    lax.fori_loop(0, PPW + 1, pair_body, 0)


_sc_gather = functools.partial(
    pl.kernel,
    out_type=[
        jax.ShapeDtypeStruct((NPAIR * CACHE, D), jnp.float32),
        jax.ShapeDtypeStruct((NPAIR * CACHE, D), jnp.float32),
        jax.ShapeDtypeStruct((NPAIR, CACHE), jnp.int32),
    ],
    mesh=plsc.VectorSubcoreMesh(core_axis_name="c", subcore_axis_name="s"),
    scratch_types=[
        pltpu.VMEM((T,), jnp.int32),
        pltpu.VMEM((SEL,), jnp.int32),
        pltpu.VMEM((SEL,), jnp.int32),
        pltpu.VMEM((2 * CACHE,), jnp.int32),
        pltpu.VMEM((2 * CACHE,), jnp.int32),
        pltpu.VMEM((3, 128, D), jnp.float32),
        pltpu.SemaphoreType.DMA,
        pltpu.SemaphoreType.DMA,
    ],
    compiler_params=pltpu.CompilerParams(needs_layout_passes=False),
)(_sc_body)


def kernel(attn_score_cache, key_cache, value_cache):
    B, H, Q, T_ = attn_score_cache.shape
    attn = attn_score_cache.reshape(B * H, Q, T_)
    hh_bits = _tc_sum(attn)
    kf = key_cache.reshape(B * H * T_, D)
    vf = value_cache.reshape(B * H * T_, D)
    ko, vo, hn = _sc_gather(hh_bits, kf, vf)
    hn_f = lax.bitcast_convert_type(hn, jnp.float32)
    return (
        ko.reshape(B, H, CACHE, D),
        vo.reshape(B, H, CACHE, D),
        hn_f.reshape(B, H, CACHE),
    )


# speedup vs baseline: 1.0676x; 1.0676x over previous
import functools

import jax
import jax.numpy as jnp
from jax import lax
from jax.experimental import pallas as pl
from jax.experimental.pallas import tpu as pltpu
from jax.experimental.pallas import tpu_sc as plsc

HH = 512
RECENT = 512
CACHE = HH + RECENT
T = 4096
SEL = T - RECENT
D = 128
NPAIR = 256
NC, NS = 2, 16
NW = NC * NS
PPW = NPAIR // NW


ROWS_PER_STEP = 32


def _sum_body(attn_ref, hh_ref):
    x = attn_ref[...]
    hh = jnp.sum(x, axis=1)
    hh_ref[...] = lax.bitcast_convert_type(hh, jnp.int32)


def _tc_sum(attn):
    return pl.pallas_call(
        _sum_body,
        grid=(NPAIR // ROWS_PER_STEP,),
        in_specs=[pl.BlockSpec((ROWS_PER_STEP, 8, T), lambda i: (i, 0, 0))],
        out_specs=pl.BlockSpec((ROWS_PER_STEP, T), lambda i: (i, 0)),
        out_shape=jax.ShapeDtypeStruct((NPAIR, T), jnp.int32),
    )(attn)


def _sc_body(hh_hbm, k_hbm, v_hbm, ko_hbm, vo_hbm, hn_hbm,
             row_v, ca_v, cb_v, idx_v, hn_v, buf_v, gsem, osem):
    wid = lax.axis_index("s") * NC + lax.axis_index("c")
    lane = lax.iota(jnp.int32, 16)

    def pair_body(j, _):
        p = wid * PPW + j
        base = p * T
        orow = p * CACHE
        coff = 0
        pltpu.sync_copy(hh_hbm.at[p], row_v)

        NB = 3
        NT = 16
        gd = [pltpu.make_async_copy(
            (k_hbm if t % 2 == 0 else v_hbm).at[
                pl.ds(base + SEL + ((t % 8) // 2) * 128, 128)]
            if t < 8 else
            (k_hbm if t % 2 == 0 else v_hbm).at[
                idx_v.at[pl.ds(((t % 8) // 2) * 128, 128)]],
            buf_v.at[t % NB], gsem) for t in range(NT)]
        od = [pltpu.make_async_copy(
            buf_v.at[t % NB],
            (ko_hbm if t % 2 == 0 else vo_hbm).at[
                pl.ds(orow + ((4 if t < 8 else 0) + (t % 8) // 2) * 128,
                      128)],
            osem) for t in range(NT)]

        def step(t):
            if t >= NB:
                od[t - NB].wait()
            gd[t].start()
            if t >= 1:
                gd[t - 1].wait()
                od[t - 1].start()

        step(0)
        step(1)
        step(2)

        def mmx(i, c):
            mn, mx = c
            v = row_v[pl.ds(i * 16, 16)]
            return jnp.minimum(mn, jnp.min(v)), jnp.maximum(mx, jnp.max(v))

        mn, mx = lax.fori_loop(
            0, SEL // 16, mmx,
            (jnp.int32(0x7FFFFFFF), jnp.int32(-0x80000000)))
        rng = mx - mn

        def hib(b, h):
            return jnp.where((rng >> b) > 0, b, h)

        h = lax.fori_loop(0, 31, hib, jnp.int32(0))
        s = 30 - h
        step(3)

        def icp(i, _):
            ca_v[pl.ds(i * 16, 16)] = (row_v[pl.ds(i * 16, 16)] - mn) << s
            return 0

        lax.fori_loop(0, SEL // 16, icp, 0)
        step(4)

        def rnd(t, st):
            ncand, need, prefix = st
            bit = 30 - t
            nv = (ncand + 15) >> 4

            def c1b(i, c):
                v = ca_v[pl.ds(i * 16, 16)]
                valid = (i * 16 + lane) < ncand
                one = jnp.logical_and(((v >> bit) & 1) == 1, valid)
                return c + jnp.sum(one.astype(jnp.int32))

            cnt1 = lax.fori_loop(0, nv, c1b, jnp.int32(0))
            pick1 = cnt1 >= need
            nnext = jnp.where(pick1, cnt1, ncand - cnt1)
            need2 = jnp.where(pick1, need, need - cnt1)
            prefix2 = jnp.where(pick1, prefix | (1 << bit), prefix)

            def sb(i, c):
                v = ca_v[pl.ds(i * 16, 16)]
                valid = (i * 16 + lane) < ncand
                one = ((v >> bit) & 1) == 1
                keepl = jnp.logical_and(valid, one == pick1)
                ki = keepl.astype(jnp.int32)
                pos = c + plsc.cumsum(ki) - 1
                plsc.store_scatter(cb_v, [pos], v, mask=keepl)
                return c + jnp.sum(ki)

            lax.fori_loop(0, nv, sb, jnp.int32(0))

            def cpb(i, _):
                ca_v[pl.ds(i * 16, 16)] = cb_v[pl.ds(i * 16, 16)]
                return 0

            lax.fori_loop(0, (nnext + 15) >> 4, cpb, 0)
            return (nnext, need2, prefix2)

        st = (jnp.int32(SEL), jnp.int32(HH), jnp.int32(0))
        for ri in range(3):
            st = lax.fori_loop(2 * ri, 2 * ri + 2, rnd, st)
            step(5 + ri)
        st = lax.fori_loop(6, 31, rnd, st)
        _, rq, prefix = st
        vstar = (prefix >> s) + mn

        def cb(i, carry):
            ck, ct = carry
            v = row_v[pl.ds(i * 16, 16)]
            gt = v > vstar
            eq = v == vstar
            eqi = eq.astype(jnp.int32)
            tie_rank = ct + plsc.cumsum(eqi)
            keep = jnp.logical_or(gt, jnp.logical_and(eq, tie_rank <= rq))
            kint = keep.astype(jnp.int32)
            pos = coff + ck + plsc.cumsum(kint) - 1
            gidx = base + i * 16 + lane
            plsc.store_scatter(idx_v, [pos], gidx, mask=keep)
            plsc.store_scatter(hn_v, [pos], v, mask=keep)
            return ck + jnp.sum(kint), ct + jnp.sum(eqi)

        lax.fori_loop(0, SEL // 16, cb, (jnp.int32(0), jnp.int32(0)))
        step(8)

        def rb(i, _):
            off = i * 16
            hn_v[pl.ds(coff + HH + off, 16)] = row_v[pl.ds(SEL + off, 16)]
            return 0

        lax.fori_loop(0, RECENT // 16, rb, 0)
        step(9)
        pltpu.sync_copy(hn_v.at[pl.ds(0, CACHE)], hn_hbm.at[p])
        for t in range(10, NT):
            step(t)
        gd[NT - 1].wait()
        od[NT - 1].start()
        for t in range(NT - NB, NT):
            od[t].wait()
        return 0

    lax.fori_loop(0, PPW, pair_body, 0)


_sc_gather = functools.partial(
    pl.kernel,
    out_type=[
        jax.ShapeDtypeStruct((NPAIR * CACHE, D), jnp.float32),
        jax.ShapeDtypeStruct((NPAIR * CACHE, D), jnp.float32),
        jax.ShapeDtypeStruct((NPAIR, CACHE), jnp.int32),
    ],
    mesh=plsc.VectorSubcoreMesh(core_axis_name="c", subcore_axis_name="s"),
    scratch_types=[
        pltpu.VMEM((T,), jnp.int32),
        pltpu.VMEM((SEL,), jnp.int32),
        pltpu.VMEM((SEL,), jnp.int32),
        pltpu.VMEM((CACHE,), jnp.int32),
        pltpu.VMEM((CACHE,), jnp.int32),
        pltpu.VMEM((3, 128, D), jnp.float32),
        pltpu.SemaphoreType.DMA,
        pltpu.SemaphoreType.DMA,
    ],
    compiler_params=pltpu.CompilerParams(needs_layout_passes=False),
)(_sc_body)


def kernel(attn_score_cache, key_cache, value_cache):
    B, H, Q, T_ = attn_score_cache.shape
    attn = attn_score_cache.reshape(B * H, Q, T_)
    hh_bits = _tc_sum(attn)
    kf = key_cache.reshape(B * H * T_, D)
    vf = value_cache.reshape(B * H * T_, D)
    ko, vo, hn = _sc_gather(hh_bits, kf, vf)
    hn_f = lax.bitcast_convert_type(hn, jnp.float32)
    return (
        ko.reshape(B, H, CACHE, D),
        vo.reshape(B, H, CACHE, D),
        hn_f.reshape(B, H, CACHE),
    )
